# baseline (device time: 9490 ns/iter reference)
import jax
import jax.numpy as jnp
from jax import lax
from jax.experimental import pallas as pl
from jax.experimental.pallas import tpu as pltpu

N_DEV = 4
N_CHUNK = 2


def kernel(x):
    m, n = x.shape
    hm = m // N_CHUNK

    def body(x_ref, out_ref, comm_ref, send_sems, recv_sems, ready_sems):
        my = lax.axis_index("i")
        left = (my - 1) % N_DEV
        right = (my + 1) % N_DEV
        diag = (my + 2) % N_DEV

        barrier_sem = pltpu.get_barrier_semaphore()
        pl.semaphore_signal(
            ready_sems.at[0], inc=1, device_id=(left,),
            device_id_type=pl.DeviceIdType.MESH,
        )
        pl.semaphore_signal(
            ready_sems.at[1], inc=1, device_id=(right,),
            device_id_type=pl.DeviceIdType.MESH,
        )
        pl.semaphore_signal(
            barrier_sem, inc=1, device_id=(diag,),
            device_id_type=pl.DeviceIdType.MESH,
        )

        comm_ref[3] = x_ref[:, :].astype(jnp.bfloat16)

        def push(dst_slot, target, h):
            r = pltpu.make_async_remote_copy(
                src_ref=comm_ref.at[3, pl.ds(h * hm, hm)],
                dst_ref=comm_ref.at[dst_slot, pl.ds(h * hm, hm)],
                send_sem=send_sems.at[dst_slot, h],
                recv_sem=recv_sems.at[dst_slot, h],
                device_id=(target,),
                device_id_type=pl.DeviceIdType.MESH,
            )
            r.start()
            return r

        pl.semaphore_wait(ready_sems.at[0], 1)
        r_r0 = push(0, right, 0)
        r_r1 = push(0, right, 1)
        pl.semaphore_wait(ready_sems.at[1], 1)
        r_l0 = push(2, left, 0)
        r_l1 = push(2, left, 1)
        pl.semaphore_wait(barrier_sem, 1)
        r_d0 = push(1, diag, 0)
        r_d1 = push(1, diag, 1)

        top = pl.ds(0, hm)
        bot = pl.ds(hm, hm)

        r_r0.wait_recv()
        acc_t = x_ref[top, :] + comm_ref[0, top, :].astype(jnp.float32)
        r_l0.wait_recv()
        acc_t = acc_t + comm_ref[2, top, :].astype(jnp.float32)
        r_d0.wait_recv()
        out_ref[top, :] = acc_t + comm_ref[1, top, :].astype(jnp.float32)

        r_r1.wait_recv()
        acc_b = x_ref[bot, :] + comm_ref[0, bot, :].astype(jnp.float32)
        r_l1.wait_recv()
        acc_b = acc_b + comm_ref[2, bot, :].astype(jnp.float32)
        r_d1.wait_recv()
        out_ref[bot, :] = acc_b + comm_ref[1, bot, :].astype(jnp.float32)

        for r in (r_r0, r_r1, r_l0, r_l1, r_d0, r_d1):
            r.wait_send()

    return pl.pallas_call(
        body,
        out_shape=jax.ShapeDtypeStruct((m, n), jnp.float32),
        in_specs=[pl.BlockSpec(memory_space=pltpu.VMEM)],
        out_specs=pl.BlockSpec(memory_space=pltpu.VMEM),
        scratch_shapes=[
            pltpu.VMEM((N_DEV, m, n), jnp.bfloat16),
            pltpu.SemaphoreType.DMA((3, N_CHUNK)),
            pltpu.SemaphoreType.DMA((3, N_CHUNK)),
            pltpu.SemaphoreType.REGULAR((2,)),
        ],
        compiler_params=pltpu.CompilerParams(collective_id=0),
    )(x)


# device time: 9400 ns/iter; 1.0096x vs baseline; 1.0096x over previous
import jax
import jax.numpy as jnp
from jax import lax
from jax.experimental import pallas as pl
from jax.experimental.pallas import tpu as pltpu

N_DEV = 4


def kernel(x):
    m, n = x.shape

    def body(x_ref, out_ref, comm_ref, send_sems, recv_sems, ready_sems):
        my = lax.axis_index("i")
        left = (my - 1) % N_DEV
        right = (my + 1) % N_DEV
        diag = (my + 2) % N_DEV

        barrier_sem = pltpu.get_barrier_semaphore()
        pl.semaphore_signal(
            ready_sems.at[0], inc=1, device_id=(left,),
            device_id_type=pl.DeviceIdType.MESH,
        )
        pl.semaphore_signal(
            ready_sems.at[1], inc=1, device_id=(right,),
            device_id_type=pl.DeviceIdType.MESH,
        )
        pl.semaphore_signal(
            barrier_sem, inc=1, device_id=(diag,),
            device_id_type=pl.DeviceIdType.MESH,
        )

        comm_ref[3] = x_ref[:, :].astype(jnp.bfloat16)

        def push(dst_slot, target):
            r = pltpu.make_async_remote_copy(
                src_ref=comm_ref.at[3],
                dst_ref=comm_ref.at[dst_slot],
                send_sem=send_sems.at[dst_slot],
                recv_sem=recv_sems.at[dst_slot],
                device_id=(target,),
                device_id_type=pl.DeviceIdType.MESH,
            )
            r.start()
            return r

        pl.semaphore_wait(ready_sems.at[0], 1)
        r_right = push(0, right)
        pl.semaphore_wait(ready_sems.at[1], 1)
        r_left = push(2, left)
        pl.semaphore_wait(barrier_sem, 1)
        r_diag = push(1, diag)

        r_right.wait_recv()
        acc = x_ref[:, :] + comm_ref[0].astype(jnp.float32)
        r_left.wait_recv()
        acc = acc + comm_ref[2].astype(jnp.float32)
        r_diag.wait_recv()
        out_ref[:, :] = acc + comm_ref[1].astype(jnp.float32)

        r_right.wait_send()
        r_left.wait_send()
        r_diag.wait_send()

    return pl.pallas_call(
        body,
        out_shape=jax.ShapeDtypeStruct((m, n), jnp.float32),
        in_specs=[pl.BlockSpec(memory_space=pltpu.VMEM)],
        out_specs=pl.BlockSpec(memory_space=pltpu.VMEM),
        scratch_shapes=[
            pltpu.VMEM((N_DEV, m, n), jnp.bfloat16),
            pltpu.SemaphoreType.DMA((3,)),
            pltpu.SemaphoreType.DMA((3,)),
            pltpu.SemaphoreType.REGULAR((2,)),
        ],
        compiler_params=pltpu.CompilerParams(collective_id=0),
    )(x)
